# baseline (speedup 1.0000x reference)
import functools

import jax
import jax.numpy as jnp
from jax import lax
from jax.experimental import pallas as pl
from jax.experimental.pallas import tpu as pltpu
from jax.experimental.pallas import tpu_sc as plsc

L = 16
NC, NS = 2, 16
NW = NC * NS
HW = 256
B_DIM, C_DIM = 64, 768
CT = C_DIM // 128
UNITS = B_DIM * CT
UNITS_PER_TILE = UNITS // NW
NCLS = 21841


def _vs(v, desc):
    return plsc.sort_key_val(v, v, descending=desc)[0]


def _bmerge(vs, desc):
    m = len(vs)
    if m == 1:
        return [_vs(vs[0], desc)]
    half = m // 2
    lo, hi = [], []
    for i in range(half):
        a, b = vs[i], vs[i + half]
        mn = jnp.minimum(a, b)
        mx = jnp.maximum(a, b)
        lo.append(mx if desc else mn)
        hi.append(mn if desc else mx)
    return _bmerge(lo, desc) + _bmerge(hi, desc)


def _bsort(vs, desc):
    if len(vs) == 1:
        return [_vs(vs[0], desc)]
    half = len(vs) // 2
    return _bmerge(_bsort(vs[:half], False) + _bsort(vs[half:], True), desc)


def _row_median(vs):
    a = _bsort(vs[:8], False)
    b = _bsort(vs[8:], True)
    u = jnp.maximum(a[0], b[0])
    for i in range(1, 8):
        u = jnp.minimum(u, jnp.maximum(a[i], b[i]))
    return _vs(u, False)


def _median_sc(xt):
    mesh = plsc.VectorSubcoreMesh(
        core_axis_name="c", subcore_axis_name="s",
        num_cores=NC, num_subcores=NS)

    BP = 129

    @functools.partial(
        pl.kernel,
        out_type=jax.ShapeDtypeStruct((B_DIM, C_DIM), jnp.float32),
        mesh=mesh,
        scratch_types=[
            pltpu.VMEM((HW, BP), jnp.float32),
            pltpu.VMEM((128,), jnp.float32),
            pltpu.SemaphoreType.DMA,
        ],
        compiler_params=pltpu.CompilerParams(
            needs_layout_passes=False, use_tc_tiling_on_sc=True),
    )
    def med_kernel(x_hbm, out_hbm, buf0, medv, sem0):
        wid = lax.axis_index("s") * NC + lax.axis_index("c")
        base_u = wid * UNITS_PER_TILE
        iota = lax.iota(jnp.int32, L)
        mask0 = iota == 0

        def issue(u, buf, sem):
            b = u // CT
            ct = u % CT
            pltpu.async_copy(
                x_hbm.at[b, :, pl.ds(ct * 128, 128)],
                buf.at[:, pl.ds(0, 128)], sem)

        def wait(buf, sem):
            pltpu.make_async_copy(
                x_hbm.at[0, :, pl.ds(0, 128)],
                buf.at[:, pl.ds(0, 128)], sem).wait()

        def process(buf, u):
            b = u // CT
            ct = u % CT

            def ch_body(c, cc):
                cidx = jnp.broadcast_to(c, (L,)).astype(jnp.int32)
                vs = [plsc.load_gather(buf, [iota + (16 * j), cidx])
                      for j in range(16)]
                med16 = _row_median(vs)
                plsc.store_scatter(medv, [cidx], med16, mask=mask0)
                return cc

            lax.fori_loop(0, 128, ch_body, 0)
            pltpu.sync_copy(medv, out_hbm.at[b, pl.ds(ct * 128, 128)])

        def unit_body(k, carry):
            u = base_u + k
            issue(u, buf0, sem0)
            wait(buf0, sem0)
            process(buf0, u)
            return carry

        lax.fori_loop(0, UNITS_PER_TILE, unit_body, 0)

    return med_kernel(xt)


def _linear_tc(med2, W, b2):
    BN = 2048

    def mm_kernel(med_ref, w_ref, b_ref, o_ref):
        o_ref[...] = lax.dot_general(
            med_ref[...], w_ref[...], (((1,), (1,)), ((), ())),
            preferred_element_type=jnp.float32) + b_ref[...]

    return pl.pallas_call(
        mm_kernel,
        grid=(pl.cdiv(NCLS, BN),),
        in_specs=[
            pl.BlockSpec((B_DIM, C_DIM), lambda i: (0, 0)),
            pl.BlockSpec((BN, C_DIM), lambda i: (i, 0)),
            pl.BlockSpec((1, BN), lambda i: (0, i)),
        ],
        out_specs=pl.BlockSpec((B_DIM, BN), lambda i: (0, i)),
        out_shape=jax.ShapeDtypeStruct((B_DIM, NCLS), jnp.float32),
    )(med2, W, b2)


def kernel(x, W, b):
    xt = x.transpose(0, 2, 3, 1).reshape(B_DIM, HW, C_DIM)
    med = _median_sc(xt)
    return _linear_tc(med, W, b.reshape(1, NCLS))


# speedup vs baseline: 1.1258x; 1.1258x over previous
import functools

import jax
import jax.numpy as jnp
from jax import lax
from jax.experimental import pallas as pl
from jax.experimental.pallas import tpu as pltpu
from jax.experimental.pallas import tpu_sc as plsc

L = 16
NC, NS = 2, 16
NW = NC * NS
HW = 256
B_DIM, C_DIM = 64, 768
CT = C_DIM // 128
UNITS = B_DIM * CT
UNITS_PER_TILE = UNITS // NW
NCLS = 21841


def _vs(v, desc):
    return plsc.sort_key_val(v, v, descending=desc)[0]


def _bmerge(vs, desc):
    m = len(vs)
    if m == 1:
        return [_vs(vs[0], desc)]
    half = m // 2
    lo, hi = [], []
    for i in range(half):
        a, b = vs[i], vs[i + half]
        mn = jnp.minimum(a, b)
        mx = jnp.maximum(a, b)
        lo.append(mx if desc else mn)
        hi.append(mn if desc else mx)
    return _bmerge(lo, desc) + _bmerge(hi, desc)


def _bsort(vs, desc):
    if len(vs) == 1:
        return [_vs(vs[0], desc)]
    half = len(vs) // 2
    return _bmerge(_bsort(vs[:half], False) + _bsort(vs[half:], True), desc)


def _row_median(vs):
    a = _bsort(vs[:8], False)
    b = _bsort(vs[8:], True)
    u = jnp.maximum(a[0], b[0])
    for i in range(1, 8):
        u = jnp.minimum(u, jnp.maximum(a[i], b[i]))
    return _vs(u, False)


def _median_sc(xt):
    mesh = plsc.VectorSubcoreMesh(
        core_axis_name="c", subcore_axis_name="s",
        num_cores=NC, num_subcores=NS)

    BP = 129

    @functools.partial(
        pl.kernel,
        out_type=jax.ShapeDtypeStruct((B_DIM, C_DIM), jnp.float32),
        mesh=mesh,
        scratch_types=[
            pltpu.VMEM((HW, BP), jnp.float32),
            pltpu.VMEM((128,), jnp.float32),
            pltpu.SemaphoreType.DMA,
        ],
        compiler_params=pltpu.CompilerParams(
            needs_layout_passes=False, use_tc_tiling_on_sc=True),
    )
    def med_kernel(x_hbm, out_hbm, buf0, medv, sem0):
        wid = lax.axis_index("s") * NC + lax.axis_index("c")
        base_u = wid * UNITS_PER_TILE
        iota = lax.iota(jnp.int32, L)
        mask0 = iota == 0

        def issue(u, buf, sem):
            b = u // CT
            ct = u % CT
            pltpu.async_copy(
                x_hbm.at[b, :, pl.ds(ct * 128, 128)],
                buf.at[:, pl.ds(0, 128)], sem)

        def wait(buf, sem):
            pltpu.make_async_copy(
                x_hbm.at[0, :, pl.ds(0, 128)],
                buf.at[:, pl.ds(0, 128)], sem).wait()

        def process(buf, u):
            b = u // CT
            ct = u % CT

            @plsc.parallel_loop(0, 128, 1, unroll=2)
            def ch_body(c):
                cidx = jnp.broadcast_to(c, (L,)).astype(jnp.int32)
                vs = [plsc.load_gather(buf, [iota + (16 * j), cidx])
                      for j in range(16)]
                med16 = _row_median(vs)
                plsc.store_scatter(medv, [cidx], med16, mask=mask0)
            pltpu.sync_copy(medv, out_hbm.at[b, pl.ds(ct * 128, 128)])

        def unit_body(k, carry):
            u = base_u + k
            issue(u, buf0, sem0)
            wait(buf0, sem0)
            process(buf0, u)
            return carry

        lax.fori_loop(0, UNITS_PER_TILE, unit_body, 0)

    return med_kernel(xt)


def _linear_tc(med2, W, b2):
    BN = 2048

    def mm_kernel(med_ref, w_ref, b_ref, o_ref):
        o_ref[...] = lax.dot_general(
            med_ref[...], w_ref[...], (((1,), (1,)), ((), ())),
            preferred_element_type=jnp.float32) + b_ref[...]

    return pl.pallas_call(
        mm_kernel,
        grid=(pl.cdiv(NCLS, BN),),
        in_specs=[
            pl.BlockSpec((B_DIM, C_DIM), lambda i: (0, 0)),
            pl.BlockSpec((BN, C_DIM), lambda i: (i, 0)),
            pl.BlockSpec((1, BN), lambda i: (0, i)),
        ],
        out_specs=pl.BlockSpec((B_DIM, BN), lambda i: (0, i)),
        out_shape=jax.ShapeDtypeStruct((B_DIM, NCLS), jnp.float32),
    )(med2, W, b2)


def kernel(x, W, b):
    xt = x.transpose(0, 2, 3, 1).reshape(B_DIM, HW, C_DIM)
    med = _median_sc(xt)
    return _linear_tc(med, W, b.reshape(1, NCLS))


# speedup vs baseline: 1.1833x; 1.0511x over previous
import functools

import jax
import jax.numpy as jnp
from jax import lax
from jax.experimental import pallas as pl
from jax.experimental.pallas import tpu as pltpu
from jax.experimental.pallas import tpu_sc as plsc

L = 16
NC, NS = 2, 16
NW = NC * NS
HW = 256
B_DIM, C_DIM = 64, 768
CT = C_DIM // 128
UNITS = B_DIM * CT
UNITS_PER_TILE = UNITS // NW
NCLS = 21841


def _vs(v, desc):
    return plsc.sort_key_val(v, v, descending=desc)[0]


def _bmerge(vs, desc):
    m = len(vs)
    if m == 1:
        return [_vs(vs[0], desc)]
    half = m // 2
    lo, hi = [], []
    for i in range(half):
        a, b = vs[i], vs[i + half]
        mn = jnp.minimum(a, b)
        mx = jnp.maximum(a, b)
        lo.append(mx if desc else mn)
        hi.append(mn if desc else mx)
    return _bmerge(lo, desc) + _bmerge(hi, desc)


def _bsort(vs, desc):
    if len(vs) == 1:
        return [_vs(vs[0], desc)]
    half = len(vs) // 2
    return _bmerge(_bsort(vs[:half], False) + _bsort(vs[half:], True), desc)


def _row_median(vs):
    a = _bsort(vs[:8], False)
    b = _bsort(vs[8:], True)
    u = jnp.maximum(a[0], b[0])
    for i in range(1, 8):
        u = jnp.minimum(u, jnp.maximum(a[i], b[i]))
    return _vs(u, False)


def _median_sc(xt):
    mesh = plsc.VectorSubcoreMesh(
        core_axis_name="c", subcore_axis_name="s",
        num_cores=NC, num_subcores=NS)

    @functools.partial(
        pl.kernel,
        out_type=jax.ShapeDtypeStruct((B_DIM, C_DIM), jnp.float32),
        mesh=mesh,
        scratch_types=[
            pltpu.VMEM((HW, 128), jnp.float32),
            pltpu.VMEM((HW, 128), jnp.float32),
            pltpu.VMEM((128,), jnp.float32),
            pltpu.SemaphoreType.DMA,
            pltpu.SemaphoreType.DMA,
        ],
        compiler_params=pltpu.CompilerParams(
            needs_layout_passes=False, use_tc_tiling_on_sc=True),
    )
    def med_kernel(x_hbm, out_hbm, buf0, buf1, medv, sem0, sem1):
        wid = lax.axis_index("s") * NC + lax.axis_index("c")
        base_u = wid * UNITS_PER_TILE
        iota = lax.iota(jnp.int32, L)
        mask0 = iota == 0

        def issue(u, buf, sem):
            b = u // CT
            ct = u % CT
            pltpu.async_copy(
                x_hbm.at[b, :, pl.ds(ct * 128, 128)], buf, sem)

        def wait(buf, sem):
            pltpu.make_async_copy(
                x_hbm.at[0, :, pl.ds(0, 128)], buf, sem).wait()

        def process(buf, u):
            b = u // CT
            ct = u % CT

            @plsc.parallel_loop(0, 128, 1, unroll=2)
            def ch_body(c):
                cidx = jnp.broadcast_to(c, (L,)).astype(jnp.int32)
                vs = [plsc.load_gather(buf, [iota + (16 * j), cidx])
                      for j in range(16)]
                med16 = _row_median(vs)
                plsc.store_scatter(medv, [cidx], med16, mask=mask0)
            pltpu.sync_copy(medv, out_hbm.at[b, pl.ds(ct * 128, 128)])

        issue(base_u, buf0, sem0)

        def pair_body(k2, carry):
            u0 = base_u + 2 * k2
            issue(u0 + 1, buf1, sem1)
            wait(buf0, sem0)
            process(buf0, u0)

            @pl.when(k2 < UNITS_PER_TILE // 2 - 1)
            def _():
                issue(u0 + 2, buf0, sem0)

            wait(buf1, sem1)
            process(buf1, u0 + 1)
            return carry

        lax.fori_loop(0, UNITS_PER_TILE // 2, pair_body, 0)

    return med_kernel(xt)


def _linear_tc(med2, W, b2):
    BN = 2048

    def mm_kernel(med_ref, w_ref, b_ref, o_ref):
        o_ref[...] = lax.dot_general(
            med_ref[...], w_ref[...], (((1,), (1,)), ((), ())),
            preferred_element_type=jnp.float32) + b_ref[...]

    return pl.pallas_call(
        mm_kernel,
        grid=(pl.cdiv(NCLS, BN),),
        in_specs=[
            pl.BlockSpec((B_DIM, C_DIM), lambda i: (0, 0)),
            pl.BlockSpec((BN, C_DIM), lambda i: (i, 0)),
            pl.BlockSpec((1, BN), lambda i: (0, i)),
        ],
        out_specs=pl.BlockSpec((B_DIM, BN), lambda i: (0, i)),
        out_shape=jax.ShapeDtypeStruct((B_DIM, NCLS), jnp.float32),
    )(med2, W, b2)


def kernel(x, W, b):
    xt = x.transpose(0, 2, 3, 1).reshape(B_DIM, HW, C_DIM)
    med = _median_sc(xt)
    return _linear_tc(med, W, b.reshape(1, NCLS))
